# baseline (speedup 1.0000x reference)
import functools

import jax
import jax.numpy as jnp
from jax import lax
from jax.experimental import pallas as pl
from jax.experimental.pallas import tpu as pltpu
from jax.experimental.pallas import tpu_sc as plsc

B, T, C = 4, 2048, 768
N = B * T
NC, NS = 2, 16
NW = NC * NS
R = 32
TPW = T // NW
NTC = TPW // R
NCH = NTC * B
LPR = C // 16
NG = 4

_mesh = plsc.VectorSubcoreMesh(
    core_axis_name="c", subcore_axis_name="s", num_cores=NC, num_subcores=NS
)


@functools.partial(
    pl.kernel,
    out_type=jax.ShapeDtypeStruct((N, C), jnp.float32),
    mesh=_mesh,
    scratch_types=[
        pltpu.VMEM((B, NTC, R), jnp.int32),
        pltpu.VMEM((R, C), jnp.float32),
        pltpu.VMEM((R, C), jnp.float32),
        pltpu.VMEM((R, C), jnp.float32),
        pltpu.VMEM((R, C), jnp.float32),
        pltpu.VMEM((R, C), jnp.float32),
        pltpu.SemaphoreType.DMA,
        pltpu.SemaphoreType.DMA,
        pltpu.SemaphoreType.DMA,
        pltpu.SemaphoreType.DMA,
        pltpu.SemaphoreType.DMA,
        pltpu.SemaphoreType.DMA,
        pltpu.SemaphoreType.DMA,
        pltpu.SemaphoreType.DMA,
        pltpu.SemaphoreType.DMA,
        pltpu.SemaphoreType.DMA,
        pltpu.SemaphoreType.DMA,
        pltpu.SemaphoreType.DMA,
        pltpu.SemaphoreType.DMA,
    ],
)
def _encode(idx_hbm, wte_hbm, wpe_hbm, out_hbm,
            idx_v, g0, g1, g2, g3, pbuf,
            gs0, gs1, gs2, gs3, gb0, gb1, gb2, gb3,
            wsem, os0, os1, os2, os3):
    wid = lax.axis_index("s") * NC + lax.axis_index("c")
    t0 = wid * TPW
    pltpu.sync_copy(idx_hbm.at[:, pl.ds(wid * NTC, NTC)], idx_v)

    gbufs = (g0, g1, g2, g3)
    gsems, osems = (gs0, gs1, gs2, gs3), (os0, os1, os2, os3)
    gsemsB = (gb0, gb1, gb2, gb3)
    w_h = pltpu.async_copy(wpe_hbm.at[pl.ds(t0, R)], pbuf, wsem)
    g_h = [None] * NG
    o_h = [None] * NG

    H = R // 2

    def start(ch):
        b = ch % NG
        tc, batch = ch // B, ch % B
        g_h[b] = (
            pltpu.async_copy(
                wte_hbm.at[idx_v.at[batch, tc, pl.ds(0, H)]],
                gbufs[b].at[pl.ds(0, H)], gsems[b]
            ),
            pltpu.async_copy(
                wte_hbm.at[idx_v.at[batch, tc, pl.ds(H, H)]],
                gbufs[b].at[pl.ds(H, H)], gsemsB[b]
            ),
        )

    def issue_out(ch):
        pb = ch % NG
        tc, batch = ch // B, ch % B
        o_h[pb] = pltpu.async_copy(
            gbufs[pb], out_hbm.at[pl.ds(batch * T + t0 + tc * R, R)], osems[pb]
        )

    start(0)
    start(1)
    for ch in range(NCH):
        tc = ch // B
        b = ch % NG
        if ch > 0:
            issue_out(ch - 1)
        if ch > 0 and ch % B == 0:
            w_h = pltpu.async_copy(
                wpe_hbm.at[pl.ds(t0 + tc * R, R)], pbuf, wsem
            )
        g_h[b][0].wait()
        if ch + 2 < NCH:
            nb = (ch + 2) % NG
            if o_h[nb] is not None:
                o_h[nb].wait()
            start(ch + 2)
        if w_h is not None:
            w_h.wait()
            w_h = None
        gbuf = gbufs[b]

        def add_rows(i, _):
            for j in range(LPR):
                sl = pl.ds(j * 16, 16)
                plsc.addupdate(gbuf.at[i, sl], pbuf[i, sl])
            return _

        lax.fori_loop(0, H, add_rows, None)
        g_h[b][1].wait()
        lax.fori_loop(H, R, add_rows, None)
    issue_out(NCH - 1)
    for h in o_h:
        if h is not None:
            h.wait()


def kernel(idx, wte, wpe):
    idx_r = idx.astype(jnp.int32).reshape(B, T // R, R)
    out = _encode(idx_r, wte, wpe)
    return out.reshape(B, T, C)


# speedup vs baseline: 2.2987x; 2.2987x over previous
import functools

import jax
import jax.numpy as jnp
from jax import lax
from jax.experimental import pallas as pl
from jax.experimental.pallas import tpu as pltpu
from jax.experimental.pallas import tpu_sc as plsc

B, T, C = 4, 2048, 768
N = B * T
NC, NS = 2, 16
NW = NC * NS
R = 32
TPW = T // NW
NTC = TPW // R
NCH = NTC * B
LPR = C // 16
NG = 4

_mesh = plsc.VectorSubcoreMesh(
    core_axis_name="c", subcore_axis_name="s", num_cores=NC, num_subcores=NS
)


@functools.partial(
    pl.kernel,
    out_type=jax.ShapeDtypeStruct((N, C), jnp.float32),
    mesh=_mesh,
    scratch_types=[
        pltpu.VMEM((B, NTC, R), jnp.int32),
        pltpu.VMEM((R, C), jnp.float32),
        pltpu.VMEM((R, C), jnp.float32),
        pltpu.VMEM((R, C), jnp.float32),
        pltpu.VMEM((R, C), jnp.float32),
        pltpu.VMEM((R, C), jnp.float32),
        pltpu.SemaphoreType.DMA,
        pltpu.SemaphoreType.DMA,
        pltpu.SemaphoreType.DMA,
        pltpu.SemaphoreType.DMA,
        pltpu.SemaphoreType.DMA,
        pltpu.SemaphoreType.DMA,
        pltpu.SemaphoreType.DMA,
        pltpu.SemaphoreType.DMA,
        pltpu.SemaphoreType.DMA,
    ],
)
def _encode(idx_hbm, wte_hbm, wpe_hbm, out_hbm,
            idx_v, g0, g1, g2, g3, pbuf,
            gs0, gs1, gs2, gs3, wsem, os0, os1, os2, os3):
    wid = lax.axis_index("s") * NC + lax.axis_index("c")
    t0 = wid * TPW
    pltpu.sync_copy(idx_hbm.at[:, pl.ds(wid * NTC, NTC)], idx_v)

    gbufs = (g0, g1, g2, g3)
    gsems, osems = (gs0, gs1, gs2, gs3), (os0, os1, os2, os3)
    w_h = pltpu.async_copy(wpe_hbm.at[pl.ds(t0, R)], pbuf, wsem)
    g_h = [None] * NG
    o_h = [None] * NG

    def start(ch):
        b = ch % NG
        tc, batch = ch // B, ch % B
        g_h[b] = pltpu.async_copy(
            wte_hbm.at[idx_v.at[batch, tc]], gbufs[b], gsems[b]
        )

    def issue_out(ch):
        pb = ch % NG
        tc, batch = ch // B, ch % B
        o_h[pb] = pltpu.async_copy(
            gbufs[pb], out_hbm.at[pl.ds(batch * T + t0 + tc * R, R)], osems[pb]
        )

    start(0)
    start(1)
    for ch in range(NCH):
        tc = ch // B
        b = ch % NG
        if ch > 0:
            issue_out(ch - 1)
        if ch > 0 and ch % B == 0:
            w_h = pltpu.async_copy(
                wpe_hbm.at[pl.ds(t0 + tc * R, R)], pbuf, wsem
            )
        g_h[b].wait()
        if ch + 2 < NCH:
            nb = (ch + 2) % NG
            if o_h[nb] is not None:
                o_h[nb].wait()
            start(ch + 2)
        if w_h is not None:
            w_h.wait()
            w_h = None
        gbuf = gbufs[b]

        @plsc.parallel_loop(0, R, step=1, unroll=2)
        def add_rows(i):
            for j in range(LPR):
                sl = pl.ds(j * 16, 16)
                plsc.addupdate(gbuf.at[i, sl], pbuf[i, sl])
    issue_out(NCH - 1)
    for h in o_h:
        if h is not None:
            h.wait()


def kernel(idx, wte, wpe):
    idx_r = idx.astype(jnp.int32).reshape(B, T // R, R)
    out = _encode(idx_r, wte, wpe)
    return out.reshape(B, T, C)
